# baseline (speedup 1.0000x reference)
import functools

import jax
import jax.numpy as jnp
from jax.experimental import pallas as pl
from jax.experimental.pallas import tpu as pltpu

N_ELEM = 4
N_EXPERTS = N_ELEM * N_ELEM
B = 128


def _moe_block_kernel(e_ref, i0_ref, i1_ref,
                      sym_ref, pairg_ref, w1_ref, b1_ref, w2_ref, b2_ref,
                      w3_ref, b3_ref, out_ref, x_ref):
    b = pl.program_id(0)
    e = e_ref[b]

    def gather_row(j, carry):
        base = b * B + j
        r0 = i0_ref[base]
        r1 = i1_ref[base]
        x_ref[pl.ds(j, 1), pl.ds(0, 384)] = sym_ref[pl.ds(r0, 1), :]
        x_ref[pl.ds(j, 1), pl.ds(384, 384)] = sym_ref[pl.ds(r1, 1), :]
        return carry

    jax.lax.fori_loop(0, B, gather_row, 0)
    x_ref[:, pl.ds(768, 16)] = pairg_ref[...]

    x = x_ref[...]
    h = jnp.dot(x, w1_ref[0], preferred_element_type=jnp.float32)
    h = h + b1_ref[pl.ds(e, 1), :]
    h = jnp.where(h > 0, h, jnp.exp(h) - 1.0)
    h = jnp.dot(h, w2_ref[0], preferred_element_type=jnp.float32)
    h = h + b2_ref[pl.ds(e, 1), :]
    h = jnp.where(h > 0, h, jnp.exp(h) - 1.0)
    w3 = w3_ref[pl.ds(e, 1)][0]
    y = jnp.dot(h, w3, preferred_element_type=jnp.float32)
    y = y + b3_ref[pl.ds(e, 1), :]
    out_ref[...] = y


def kernel(elements, connectivity, sym_features, pair_features,
           W1, b1, W2, b2, W3, b3):
    n_batch, n_conn, _ = connectivity.shape
    n_atoms = sym_features.shape[1]
    d_feat = sym_features.shape[-1]
    d_pair = pair_features.shape[-1]
    n_rows = n_batch * n_conn
    nb = n_rows // B + N_EXPERTS

    offsets = (jnp.arange(n_batch, dtype=jnp.int32) * n_atoms)[:, None, None]
    conn_f = (connectivity.astype(jnp.int32) + offsets).reshape(-1, 2)
    elem_f = elements.reshape(-1).astype(jnp.int32)
    e0 = jnp.take(elem_f, conn_f[:, 0], axis=0)
    e1 = jnp.take(elem_f, conn_f[:, 1], axis=0)
    nn_index = e0 * N_ELEM + e1

    order = jnp.argsort(nn_index).astype(jnp.int32)
    counts = jnp.bincount(nn_index, length=N_EXPERTS).astype(jnp.int32)
    cum = jnp.concatenate([jnp.zeros((1,), jnp.int32), jnp.cumsum(counts)]).astype(jnp.int32)
    nblk = (counts + B - 1) // B
    blk_cum = jnp.concatenate([jnp.zeros((1,), jnp.int32), jnp.cumsum(nblk)]).astype(jnp.int32)

    b_arr = jnp.arange(nb, dtype=jnp.int32)
    e_of_b = jnp.clip(jnp.searchsorted(blk_cum, b_arr, side='right') - 1,
                      0, N_EXPERTS - 1).astype(jnp.int32)
    k = b_arr - blk_cum[e_of_b]
    p2 = cum[e_of_b][:, None] + k[:, None] * B + jnp.arange(B, dtype=jnp.int32)[None, :]
    valid = p2 < cum[e_of_b + 1][:, None]
    rows_sorted = order[jnp.clip(p2, 0, n_rows - 1)]
    rid = jnp.where(valid, rows_sorted, 0).reshape(-1)
    i0 = jnp.take(conn_f[:, 0], rid, axis=0)
    i1 = jnp.take(conn_f[:, 1], rid, axis=0)

    p = jnp.arange(n_rows, dtype=jnp.int32)
    e_p = (jnp.searchsorted(cum, p, side='right') - 1).astype(jnp.int32)
    q = p - cum[e_p]
    slot_p = (blk_cum[e_p] + q // B) * B + q % B
    inv = jnp.zeros((n_rows,), jnp.int32).at[order].set(slot_p)

    sym_flat = sym_features.reshape(-1, d_feat)
    pair_flat = pair_features.reshape(-1, d_pair)
    d_in = W1.shape[1]
    d_h1 = W1.shape[2]
    d_h2 = W2.shape[2]
    d_out = W3.shape[2]

    pair_g = jnp.take(pair_flat, rid, axis=0)

    grid_spec = pltpu.PrefetchScalarGridSpec(
        num_scalar_prefetch=3,
        grid=(nb,),
        in_specs=[
            pl.BlockSpec(sym_flat.shape, lambda b, e, i0, i1: (0, 0)),
            pl.BlockSpec((B, d_pair), lambda b, e, i0, i1: (b, 0)),
            pl.BlockSpec((1, d_in, d_h1), lambda b, e, i0, i1: (e[b], 0, 0)),
            pl.BlockSpec(b1.shape, lambda b, e, i0, i1: (0, 0)),
            pl.BlockSpec((1, d_h1, d_h2), lambda b, e, i0, i1: (e[b], 0, 0)),
            pl.BlockSpec(b2.shape, lambda b, e, i0, i1: (0, 0)),
            pl.BlockSpec(W3.shape, lambda b, e, i0, i1: (0, 0, 0)),
            pl.BlockSpec(b3.shape, lambda b, e, i0, i1: (0, 0)),
        ],
        out_specs=pl.BlockSpec((B, d_out), lambda b, e, i0, i1: (b, 0)),
        scratch_shapes=[pltpu.VMEM((B, d_in), jnp.float32)],
    )

    y_pad = pl.pallas_call(
        _moe_block_kernel,
        grid_spec=grid_spec,
        out_shape=jax.ShapeDtypeStruct((nb * B, d_out), jnp.float32),
    )(e_of_b, i0, i1, sym_flat, pair_g, W1, b1, W2, b2, W3, b3)

    y = jnp.take(y_pad, inv, axis=0).reshape(n_batch, n_conn, d_out)
    return (elements, connectivity, y)


# speedup vs baseline: 2.9637x; 2.9637x over previous
import jax
import jax.numpy as jnp
from jax.experimental import pallas as pl
from jax.experimental.pallas import tpu as pltpu

N_ELEM = 4
N_EXPERTS = N_ELEM * N_ELEM
B = 128
ZW = 32


def _moe_block_kernel(e_ref, sym_ref, z_ref, w1_ref, b1_ref, w2_ref, b2_ref,
                      w3_ref, b3_ref, out_ref):
    e = e_ref[pl.program_id(0)]
    n_flat = sym_ref.shape[0]
    d_feat = sym_ref.shape[1]

    z = z_ref[...]
    i0c = z[:, 0:1]
    i1c = z[:, 1:2]
    pair = jax.lax.bitcast_convert_type(z[:, 8:24], jnp.float32)

    aio = jax.lax.broadcasted_iota(jnp.int32, (B, n_flat), 1)
    p0 = (aio == i0c).astype(jnp.float32)
    p1 = (aio == i1c).astype(jnp.float32)
    sym = sym_ref[...]
    f0 = jnp.dot(p0, sym, preferred_element_type=jnp.float32)
    f1 = jnp.dot(p1, sym, preferred_element_type=jnp.float32)

    w1 = w1_ref[0]
    h = (jnp.dot(f0, w1[0:d_feat], preferred_element_type=jnp.float32)
         + jnp.dot(f1, w1[d_feat:2 * d_feat], preferred_element_type=jnp.float32)
         + jnp.dot(pair, w1[2 * d_feat:], preferred_element_type=jnp.float32)
         + b1_ref[pl.ds(e, 1), :])
    h = jnp.where(h > 0, h, jnp.exp(h) - 1.0)
    h = jnp.dot(h, w2_ref[0], preferred_element_type=jnp.float32)
    h = h + b2_ref[pl.ds(e, 1), :]
    h = jnp.where(h > 0, h, jnp.exp(h) - 1.0)
    w3 = w3_ref[pl.ds(e, 1)][0]
    y = jnp.dot(h, w3, preferred_element_type=jnp.float32)
    out_ref[...] = y + b3_ref[pl.ds(e, 1), :]


def kernel(elements, connectivity, sym_features, pair_features,
           W1, b1, W2, b2, W3, b3):
    n_batch, n_conn, _ = connectivity.shape
    n_atoms = sym_features.shape[1]
    d_feat = sym_features.shape[-1]
    d_pair = pair_features.shape[-1]
    n_rows = n_batch * n_conn
    nb = n_rows // B + N_EXPERTS

    offsets = (jnp.arange(n_batch, dtype=jnp.int32) * n_atoms)[:, None, None]
    conn_f = (connectivity.astype(jnp.int32) + offsets).reshape(-1, 2)
    elem_f = elements.reshape(-1).astype(jnp.int32)
    i0_row = conn_f[:, 0]
    i1_row = conn_f[:, 1]
    e01 = jnp.take(elem_f, jnp.concatenate([i0_row, i1_row]), axis=0)
    key = e01[:n_rows] * N_ELEM + e01[n_rows:]

    onehot = (key[:, None] == jnp.arange(N_EXPERTS, dtype=jnp.int32)[None, :]
              ).astype(jnp.int32)
    csum = jnp.cumsum(onehot, axis=0)
    counts = csum[-1]
    pos = jnp.sum(onehot * csum, axis=1) - 1
    nblk = (counts + B - 1) // B
    blk_cum = jnp.concatenate(
        [jnp.zeros((1,), jnp.int32), jnp.cumsum(nblk)]).astype(jnp.int32)
    blk_base = jnp.sum(onehot * blk_cum[None, :N_EXPERTS], axis=1)
    slot = (blk_base + pos // B) * B + pos % B

    pair_flat = pair_features.reshape(-1, d_pair)
    pair_i = jax.lax.bitcast_convert_type(pair_flat, jnp.int32)
    packed = jnp.concatenate(
        [i0_row[:, None], i1_row[:, None],
         jnp.zeros((n_rows, 6), jnp.int32), pair_i,
         jnp.zeros((n_rows, ZW - 8 - d_pair), jnp.int32)], axis=1)
    z = jnp.zeros((nb * B, ZW), jnp.int32).at[slot].set(packed)

    b_arr = jnp.arange(nb, dtype=jnp.int32)
    e_of_b = jnp.clip(jnp.searchsorted(blk_cum, b_arr, side='right') - 1,
                      0, N_EXPERTS - 1).astype(jnp.int32)

    sym_flat = sym_features.reshape(-1, d_feat)
    d_in = W1.shape[1]
    d_h1 = W1.shape[2]
    d_h2 = W2.shape[2]
    d_out = W3.shape[2]

    grid_spec = pltpu.PrefetchScalarGridSpec(
        num_scalar_prefetch=1,
        grid=(nb,),
        in_specs=[
            pl.BlockSpec(sym_flat.shape, lambda b, e: (0, 0)),
            pl.BlockSpec((B, ZW), lambda b, e: (b, 0)),
            pl.BlockSpec((1, d_in, d_h1), lambda b, e: (e[b], 0, 0)),
            pl.BlockSpec(b1.shape, lambda b, e: (0, 0)),
            pl.BlockSpec((1, d_h1, d_h2), lambda b, e: (e[b], 0, 0)),
            pl.BlockSpec(b2.shape, lambda b, e: (0, 0)),
            pl.BlockSpec(W3.shape, lambda b, e: (0, 0, 0)),
            pl.BlockSpec(b3.shape, lambda b, e: (0, 0)),
        ],
        out_specs=pl.BlockSpec((B, d_out), lambda b, e: (b, 0)),
    )

    y_pad = pl.pallas_call(
        _moe_block_kernel,
        grid_spec=grid_spec,
        out_shape=jax.ShapeDtypeStruct((nb * B, d_out), jnp.float32),
    )(e_of_b, sym_flat, z, W1, b1, W2, b2, W3, b3)

    y = jnp.take(y_pad, slot, axis=0).reshape(n_batch, n_conn, d_out)
    return (elements, connectivity, y)
